# baseline (speedup 1.0000x reference)
import functools

import jax
import jax.numpy as jnp
from jax import lax
from jax.experimental import pallas as pl
from jax.experimental.pallas import tpu as pltpu
from jax.experimental.pallas import tpu_sc as plsc

_N, _D, _H, _O, _B = 100000, 128, 128, 128, 1024
_BN = 2000
_NB = _N // _BN

_CH = 128
_NFULL = _N // _CH
_REM = _N - _NFULL * _CH
_NW = 32
_STRIPE = _B // 16


def _mlp_body(x_ref, w1_ref, b1_ref, w2_ref, b2_ref, h_ref):
    x = x_ref[...]
    h = jnp.dot(x, w1_ref[...], preferred_element_type=jnp.float32)
    h = jnp.maximum(h + b1_ref[...], 0.0)
    h = jnp.dot(h, w2_ref[...], preferred_element_type=jnp.float32)
    h_ref[...] = jnp.maximum(h + b2_ref[...], 0.0)


def _mlp(x, W1, b1, W2, b2):
    return pl.pallas_call(
        _mlp_body,
        grid=(_NB,),
        in_specs=[
            pl.BlockSpec((_BN, _D), lambda g: (g, 0)),
            pl.BlockSpec((_D, _H), lambda g: (0, 0)),
            pl.BlockSpec((1, _H), lambda g: (0, 0)),
            pl.BlockSpec((_H, _H), lambda g: (0, 0)),
            pl.BlockSpec((1, _H), lambda g: (0, 0)),
        ],
        out_specs=pl.BlockSpec((_BN, _H), lambda g: (g, 0)),
        out_shape=jax.ShapeDtypeStruct((_N, _H), jnp.float32),
    )(x, W1, b1.reshape(1, _H), W2, b2.reshape(1, _H))


def _segsum_body(h_hbm, ids_hbm, zeros_hbm, out_hbm,
                 idx_v, rows_v, idx_r, rows_r, acc_sh):
    cid = lax.axis_index("c")
    sid = lax.axis_index("s")
    wid = sid * 2 + cid

    pltpu.sync_copy(zeros_hbm.at[pl.ds(sid * _STRIPE, _STRIPE)],
                    acc_sh.at[pl.ds(sid * _STRIPE, _STRIPE)])
    plsc.subcore_barrier()

    def chunk_body(t, carry):
        c = wid + _NW * t
        base = c * _CH
        pltpu.sync_copy(ids_hbm.at[pl.ds(base, _CH)], idx_v)
        pltpu.sync_copy(h_hbm.at[pl.ds(base, _CH)], rows_v)
        pltpu.sync_copy(rows_v, acc_sh.at[idx_v], add=True)
        return carry

    n_w = jnp.where(wid < (_NFULL % _NW), _NFULL // _NW + 1, _NFULL // _NW)
    lax.fori_loop(0, n_w, chunk_body, 0)

    @pl.when(wid == _NW - 1)
    def _rem():
        base = _NFULL * _CH
        pltpu.sync_copy(ids_hbm.at[pl.ds(base, _REM)], idx_r)
        pltpu.sync_copy(h_hbm.at[pl.ds(base, _REM)], rows_r)
        pltpu.sync_copy(rows_r, acc_sh.at[idx_r], add=True)

    plsc.subcore_barrier()
    out_base = cid * _B + sid * _STRIPE
    pltpu.sync_copy(acc_sh.at[pl.ds(sid * _STRIPE, _STRIPE)],
                    out_hbm.at[pl.ds(out_base, _STRIPE)])


def _segsum(h, ids, zeros):
    mesh = plsc.VectorSubcoreMesh(core_axis_name="c", subcore_axis_name="s")
    f = functools.partial(
        pl.kernel,
        mesh=mesh,
        out_type=jax.ShapeDtypeStruct((2 * _B, _H), jnp.float32),
        scratch_types=[
            pltpu.VMEM((_CH,), jnp.int32),
            pltpu.VMEM((_CH, _H), jnp.float32),
            pltpu.VMEM((_REM,), jnp.int32),
            pltpu.VMEM((_REM, _H), jnp.float32),
            pltpu.VMEM_SHARED((_B, _H), jnp.float32),
        ],
    )(_segsum_body)
    return f(h, ids, zeros)


def _out_body(p_ref, wout_ref, bout_ref, out_ref):
    acc = p_ref[0:_B, :] + p_ref[_B:2 * _B, :]
    out_ref[...] = (jnp.dot(acc, wout_ref[...],
                            preferred_element_type=jnp.float32)
                    + bout_ref[...])


def _out_layer(partials, Wout, bout):
    return pl.pallas_call(
        _out_body,
        in_specs=[
            pl.BlockSpec((2 * _B, _H), lambda: (0, 0)),
            pl.BlockSpec((_H, _O), lambda: (0, 0)),
            pl.BlockSpec((1, _O), lambda: (0, 0)),
        ],
        out_specs=pl.BlockSpec((_B, _O), lambda: (0, 0)),
        out_shape=jax.ShapeDtypeStruct((_B, _O), jnp.float32),
    )(partials, Wout, bout.reshape(1, _O))


def kernel(node_features, batch_vector, W1, b1, W2, b2, Wout, bout):
    h = _mlp(node_features, W1, b1, W2, b2)
    ids = batch_vector.astype(jnp.int32)
    zeros = jnp.zeros((_B, _H), jnp.float32)
    partials = _segsum(h, ids, zeros)
    return _out_layer(partials, Wout, bout)


# speedup vs baseline: 1.2553x; 1.2553x over previous
import functools

import jax
import jax.numpy as jnp
from jax import lax
from jax.experimental import pallas as pl
from jax.experimental.pallas import tpu as pltpu
from jax.experimental.pallas import tpu_sc as plsc

_N, _D, _H, _O, _B = 100000, 128, 128, 128, 1024
_BN = 2000
_NB = _N // _BN

_CH = 128
_NFULL = _N // _CH
_REM = _N - _NFULL * _CH
_NW = 32
_STRIPE = _B // 16


def _mlp_body(x_ref, w1_ref, b1_ref, w2_ref, b2_ref, h_ref):
    x = x_ref[...]
    h = jnp.dot(x, w1_ref[...], preferred_element_type=jnp.float32)
    h = jnp.maximum(h + b1_ref[...], 0.0)
    h = jnp.dot(h, w2_ref[...], preferred_element_type=jnp.float32)
    h_ref[...] = jnp.maximum(h + b2_ref[...], 0.0)


def _mlp(x, W1, b1, W2, b2):
    return pl.pallas_call(
        _mlp_body,
        grid=(_NB,),
        in_specs=[
            pl.BlockSpec((_BN, _D), lambda g: (g, 0)),
            pl.BlockSpec((_D, _H), lambda g: (0, 0)),
            pl.BlockSpec((1, _H), lambda g: (0, 0)),
            pl.BlockSpec((_H, _H), lambda g: (0, 0)),
            pl.BlockSpec((1, _H), lambda g: (0, 0)),
        ],
        out_specs=pl.BlockSpec((_BN, _H), lambda g: (g, 0)),
        out_shape=jax.ShapeDtypeStruct((_N, _H), jnp.float32),
    )(x, W1, b1.reshape(1, _H), W2, b2.reshape(1, _H))


_TMAX = -(-_NFULL // _NW)
_NTAIL = _NFULL % _NW


def _segsum_body(h_hbm, ids_hbm, zeros_hbm, out_hbm,
                 idx0, idx1, rows0, rows1, idx_r, rows_r, acc_sh,
                 sem0, sem1):
    cid = lax.axis_index("c")
    sid = lax.axis_index("s")
    wid = sid * 2 + cid
    idx = (idx0, idx1)
    rows = (rows0, rows1)
    sem = (sem0, sem1)

    pltpu.sync_copy(zeros_hbm.at[pl.ds(sid * _STRIPE, _STRIPE)],
                    acc_sh.at[pl.ds(sid * _STRIPE, _STRIPE)])

    def start(t, b):
        base = (wid + _NW * t) * _CH
        pltpu.async_copy(ids_hbm.at[pl.ds(base, _CH)], idx[b], sem[b])
        pltpu.async_copy(h_hbm.at[pl.ds(base, _CH)], rows[b], sem[b])

    def finish(t, b):
        base = (wid + _NW * t) * _CH
        pltpu.make_async_copy(ids_hbm.at[pl.ds(base, _CH)], idx[b],
                              sem[b]).wait()
        pltpu.make_async_copy(h_hbm.at[pl.ds(base, _CH)], rows[b],
                              sem[b]).wait()
        pltpu.sync_copy(rows[b], acc_sh.at[idx[b]], add=True)

    start(0, 0)
    plsc.subcore_barrier()

    for t in range(_TMAX):
        b = t & 1
        if t + 1 < _TMAX - 1:
            start(t + 1, 1 - b)
        elif t + 1 == _TMAX - 1:
            @pl.when(wid < _NTAIL)
            def _start_tail():
                start(_TMAX - 1, 1 - b)
        if t < _TMAX - 1:
            finish(t, b)
        else:
            @pl.when(wid < _NTAIL)
            def _finish_tail():
                finish(_TMAX - 1, b)

    @pl.when(wid == _NW - 1)
    def _rem():
        base = _NFULL * _CH
        pltpu.sync_copy(ids_hbm.at[pl.ds(base, _REM)], idx_r)
        pltpu.sync_copy(h_hbm.at[pl.ds(base, _REM)], rows_r)
        pltpu.sync_copy(rows_r, acc_sh.at[idx_r], add=True)

    plsc.subcore_barrier()
    out_base = cid * _B + sid * _STRIPE
    pltpu.sync_copy(acc_sh.at[pl.ds(sid * _STRIPE, _STRIPE)],
                    out_hbm.at[pl.ds(out_base, _STRIPE)])


def _segsum(h, ids, zeros):
    mesh = plsc.VectorSubcoreMesh(core_axis_name="c", subcore_axis_name="s")
    f = functools.partial(
        pl.kernel,
        mesh=mesh,
        out_type=jax.ShapeDtypeStruct((2 * _B, _H), jnp.float32),
        scratch_types=[
            pltpu.VMEM((_CH,), jnp.int32),
            pltpu.VMEM((_CH,), jnp.int32),
            pltpu.VMEM((_CH, _H), jnp.float32),
            pltpu.VMEM((_CH, _H), jnp.float32),
            pltpu.VMEM((_REM,), jnp.int32),
            pltpu.VMEM((_REM, _H), jnp.float32),
            pltpu.VMEM_SHARED((_B, _H), jnp.float32),
            pltpu.SemaphoreType.DMA,
            pltpu.SemaphoreType.DMA,
        ],
    )(_segsum_body)
    return f(h, ids, zeros)


def _out_body(p_ref, wout_ref, bout_ref, out_ref):
    acc = p_ref[0:_B, :] + p_ref[_B:2 * _B, :]
    out_ref[...] = (jnp.dot(acc, wout_ref[...],
                            preferred_element_type=jnp.float32)
                    + bout_ref[...])


def _out_layer(partials, Wout, bout):
    return pl.pallas_call(
        _out_body,
        in_specs=[
            pl.BlockSpec((2 * _B, _H), lambda: (0, 0)),
            pl.BlockSpec((_H, _O), lambda: (0, 0)),
            pl.BlockSpec((1, _O), lambda: (0, 0)),
        ],
        out_specs=pl.BlockSpec((_B, _O), lambda: (0, 0)),
        out_shape=jax.ShapeDtypeStruct((_B, _O), jnp.float32),
    )(partials, Wout, bout.reshape(1, _O))


def kernel(node_features, batch_vector, W1, b1, W2, b2, Wout, bout):
    h = _mlp(node_features, W1, b1, W2, b2)
    ids = batch_vector.astype(jnp.int32)
    zeros = jnp.zeros((_B, _H), jnp.float32)
    partials = _segsum(h, ids, zeros)
    return _out_layer(partials, Wout, bout)
